# baseline (speedup 1.0000x reference)
import jax
import jax.numpy as jnp
from jax.experimental import pallas as pl
from jax.experimental.pallas import tpu as pltpu

_NUM_DMAS = 8


def kernel(x, table):
    seq_len = x.shape[1]
    dim = table.shape[1]
    total = seq_len * dim
    piece = total // _NUM_DMAS
    assert piece * _NUM_DMAS == total and piece % 8 == 0

    def copy_body(table_ref, out_ref, *sems):
        handles = []
        for k in range(_NUM_DMAS):
            h = pltpu.make_async_copy(
                table_ref.at[pl.ds(k * piece, piece)],
                out_ref.at[pl.ds(k * piece, piece)],
                sems[k],
            )
            h.start()
            handles.append(h)
        for h in handles:
            h.wait()

    flat = pl.pallas_call(
        copy_body,
        in_specs=[pl.BlockSpec(memory_space=pl.ANY)],
        out_specs=pl.BlockSpec(memory_space=pl.ANY),
        scratch_shapes=[pltpu.SemaphoreType.DMA] * _NUM_DMAS,
        out_shape=jax.ShapeDtypeStruct((total,), table.dtype),
    )(table.reshape(-1))
    return flat.reshape(seq_len, dim)


# speedup vs baseline: 54.3765x; 54.3765x over previous
import jax
import jax.numpy as jnp
from jax.experimental import pallas as pl

_BLOCK_ROWS = 7168


def _copy_block(table_ref, out_ref):
    out_ref[...] = table_ref[...]


def kernel(x, table):
    seq_len = x.shape[1]
    dim = table.shape[1]
    grid = (pl.cdiv(seq_len, _BLOCK_ROWS),)
    return pl.pallas_call(
        _copy_block,
        grid=grid,
        in_specs=[pl.BlockSpec((_BLOCK_ROWS, dim), lambda i: (i, 0))],
        out_specs=pl.BlockSpec((_BLOCK_ROWS, dim), lambda i: (i, 0)),
        out_shape=jax.ShapeDtypeStruct((seq_len, dim), table.dtype),
    )(table)
